# baseline (speedup 1.0000x reference)
import functools

import jax
import jax.numpy as jnp
from jax import lax
from jax.experimental import pallas as pl
from jax.experimental.pallas import tpu as pltpu
from jax.experimental.pallas import tpu_sc as plsc

VOCAB = 100000
EMBED_DIM = 64
BATCH = 4096
HIST_LEN = 50
NC, NS = 2, 16
NW = NC * NS
BT = BATCH // NW
TPAD = BT + 1

_mesh = plsc.VectorSubcoreMesh(core_axis_name="c", subcore_axis_name="s")


@functools.partial(
    pl.kernel,
    out_type=jax.ShapeDtypeStruct((HIST_LEN, 8, NW, 8, BT), jnp.float32),
    mesh=_mesh,
    scratch_types=[
        pltpu.VMEM((BT, HIST_LEN), jnp.int32),
        pltpu.VMEM((HIST_LEN, BT), jnp.int32),
        pltpu.VMEM((10, BT, EMBED_DIM), jnp.float32),
        pltpu.VMEM((2, 8, 8, TPAD), jnp.float32),
        [pltpu.SemaphoreType.DMA] * 10,
        [pltpu.SemaphoreType.DMA] * 2,
    ],
    compiler_params=pltpu.CompilerParams(
        use_tc_tiling_on_sc=False, needs_layout_passes=False),
)
def _sc_gather(table_hbm, idx_hbm, out_hbm, idx_raw_v, idx_v, rows_v, tr_v,
               gsems, osems):
    wid = lax.axis_index("s") * NC + lax.axis_index("c")
    pltpu.sync_copy(idx_hbm.at[pl.ds(wid * BT, BT)], idx_raw_v)
    iota = jnp.arange(16, dtype=jnp.int32)
    egvecs = [(iota + 16 * c) // 8 for c in range(4)]
    esvecs = [(iota + 16 * c) % 8 for c in range(4)]

    @plsc.parallel_loop(0, HIST_LEN, unroll=2)
    def _idx_transpose(h):
        hv = jnp.full((16,), h, jnp.int32)
        for bg in range(8):
            v = plsc.load_gather(idx_raw_v, [iota + 16 * bg, hv])
            idx_v[h, pl.ds(16 * bg, 16)] = v

    for j in range(10):
        pltpu.async_copy(table_hbm.at[idx_v.at[j]], rows_v.at[j], gsems[j])

    def block(b, carry):
        for j in range(10):
            h = b * 10 + j
            t = j % 2
            pltpu.make_async_copy(
                table_hbm.at[idx_v.at[h]], rows_v.at[j], gsems[j]).wait()

            @pl.when(h >= 2)
            def _wait_out():
                pltpu.make_async_copy(
                    tr_v.at[t, :, :, pl.ds(0, BT)],
                    out_hbm.at[0, :, wid], osems[t]).wait()

            rows_j = rows_v.at[j]

            @plsc.parallel_loop(0, BT, unroll=4)
            def _transpose(r):
                rv = jnp.full((16,), r, jnp.int32)
                for c in range(4):
                    v = rows_j[r, pl.ds(16 * c, 16)]
                    plsc.store_scatter(
                        tr_v.at[t], [egvecs[c], esvecs[c], rv], v)

            pltpu.async_copy(
                tr_v.at[t, :, :, pl.ds(0, BT)],
                out_hbm.at[h, :, wid], osems[t])

            @pl.when(h < HIST_LEN - 10)
            def _prefetch():
                pltpu.async_copy(
                    table_hbm.at[idx_v.at[h + 10]], rows_v.at[j], gsems[j])
        return carry

    lax.fori_loop(0, HIST_LEN // 10, block, None)
    for t in range(2):
        pltpu.make_async_copy(
            tr_v.at[t, :, :, pl.ds(0, BT)],
            out_hbm.at[0, :, wid], osems[t]).wait()


def kernel(X, table):
    q = _sc_gather(table, X.astype(jnp.int32))
    return q.transpose(2, 4, 0, 1, 3).reshape(BATCH, HIST_LEN, EMBED_DIM)


# speedup vs baseline: 1.0679x; 1.0679x over previous
import functools

import jax
import jax.numpy as jnp
from jax import lax
from jax.experimental import pallas as pl
from jax.experimental.pallas import tpu as pltpu
from jax.experimental.pallas import tpu_sc as plsc

VOCAB = 100000
EMBED_DIM = 64
BATCH = 4096
HIST_LEN = 50
NC, NS = 2, 16
NW = NC * NS
BT = BATCH // NW
TPAD = BT + 1

_mesh = plsc.VectorSubcoreMesh(core_axis_name="c", subcore_axis_name="s")


@functools.partial(
    pl.kernel,
    out_type=jax.ShapeDtypeStruct((HIST_LEN, 8, NW, 8, BT), jnp.float32),
    mesh=_mesh,
    scratch_types=[
        pltpu.VMEM((BT, HIST_LEN), jnp.int32),
        pltpu.VMEM((HIST_LEN, BT), jnp.int32),
        pltpu.VMEM((10, BT, EMBED_DIM), jnp.float32),
        pltpu.VMEM((2, 8, 8, TPAD), jnp.float32),
        [pltpu.SemaphoreType.DMA] * 10,
        [pltpu.SemaphoreType.DMA] * 2,
    ],
    compiler_params=pltpu.CompilerParams(
        use_tc_tiling_on_sc=False, needs_layout_passes=False),
)
def _sc_gather(table_hbm, idx_hbm, out_hbm, idx_raw_v, idx_v, rows_v, tr_v,
               gsems, osems):
    wid = lax.axis_index("s") * NC + lax.axis_index("c")
    pltpu.sync_copy(idx_hbm.at[pl.ds(wid * BT, BT)], idx_raw_v)
    iota = jnp.arange(16, dtype=jnp.int32)
    egvecs = [(iota + 16 * c) // 8 for c in range(4)]
    esvecs = [(iota + 16 * c) % 8 for c in range(4)]

    @plsc.parallel_loop(0, HIST_LEN, unroll=2)
    def _idx_transpose(h):
        hv = jnp.full((16,), h, jnp.int32)
        for bg in range(8):
            v = plsc.load_gather(idx_raw_v, [iota + 16 * bg, hv])
            idx_v[h, pl.ds(16 * bg, 16)] = v + v

    for j in range(10):
        pltpu.async_copy(table_hbm.at[idx_v.at[j]], rows_v.at[j], gsems[j])

    def block(b, carry):
        for j in range(10):
            h = b * 10 + j
            t = j % 2
            pltpu.make_async_copy(
                table_hbm.at[idx_v.at[h]], rows_v.at[j], gsems[j]).wait()

            @pl.when(h >= 2)
            def _wait_out():
                pltpu.make_async_copy(
                    tr_v.at[t, :, :, pl.ds(0, BT)],
                    out_hbm.at[0, :, wid], osems[t]).wait()

            rows_j = rows_v.at[j]

            @plsc.parallel_loop(0, BT, unroll=4)
            def _transpose(r):
                rv = jnp.full((16,), r, jnp.int32)
                for c in range(4):
                    v = rows_j[r, pl.ds(16 * c, 16)]
                    plsc.store_scatter(
                        tr_v.at[t], [egvecs[c], esvecs[c], rv], v)

            pltpu.async_copy(
                tr_v.at[t, :, :, pl.ds(0, BT)],
                out_hbm.at[h, :, wid], osems[t])

            @pl.when(h < HIST_LEN - 10)
            def _prefetch():
                pltpu.async_copy(
                    table_hbm.at[idx_v.at[h + 10]], rows_v.at[j], gsems[j])
        return carry

    lax.fori_loop(0, HIST_LEN // 10, block, None)
    for t in range(2):
        pltpu.make_async_copy(
            tr_v.at[t, :, :, pl.ds(0, BT)],
            out_hbm.at[0, :, wid], osems[t]).wait()


def kernel(X, table):
    table_p = jnp.pad(table, ((0, 0), (0, EMBED_DIM))).reshape(
        2 * VOCAB, EMBED_DIM)
    q = _sc_gather(table_p, X.astype(jnp.int32))
    return q.transpose(2, 4, 0, 1, 3).reshape(BATCH, HIST_LEN, EMBED_DIM)


# speedup vs baseline: 1.0800x; 1.0114x over previous
import functools

import jax
import jax.numpy as jnp
from jax import lax
from jax.experimental import pallas as pl
from jax.experimental.pallas import tpu as pltpu
from jax.experimental.pallas import tpu_sc as plsc

VOCAB = 100000
EMBED_DIM = 64
BATCH = 4096
HIST_LEN = 50
NC, NS = 2, 16
NW = NC * NS
BT = BATCH // NW
TPAD = BT + 1

_mesh = plsc.VectorSubcoreMesh(core_axis_name="c", subcore_axis_name="s")


@functools.partial(
    pl.kernel,
    out_type=jax.ShapeDtypeStruct((HIST_LEN, 8, NW, 8, BT), jnp.float32),
    mesh=_mesh,
    scratch_types=[
        pltpu.VMEM((BT, HIST_LEN), jnp.int32),
        pltpu.VMEM((HIST_LEN, BT), jnp.int32),
        pltpu.VMEM((6, BT, EMBED_DIM), jnp.float32),
        pltpu.VMEM((6, 8, 8, TPAD), jnp.float32),
        [pltpu.SemaphoreType.DMA] * 6,
        [pltpu.SemaphoreType.DMA] * 6,
    ],
    compiler_params=pltpu.CompilerParams(
        use_tc_tiling_on_sc=False, needs_layout_passes=False),
)
def _sc_gather(table_hbm, idx_hbm, out_hbm, idx_raw_v, idx_v, rows_v, tr_v,
               gsems, osems):
    wid = lax.axis_index("s") * NC + lax.axis_index("c")
    pltpu.sync_copy(idx_hbm.at[pl.ds(wid * BT, BT)], idx_raw_v)
    iota = jnp.arange(16, dtype=jnp.int32)
    egvecs = [(iota + 16 * c) // 8 for c in range(4)]
    esvecs = [(iota + 16 * c) % 8 for c in range(4)]

    @plsc.parallel_loop(0, HIST_LEN, unroll=2)
    def _idx_transpose(h):
        hv = jnp.full((16,), h, jnp.int32)
        for bg in range(8):
            v = plsc.load_gather(idx_raw_v, [iota + 16 * bg, hv])
            idx_v[h, pl.ds(16 * bg, 16)] = v + v

    for j in range(6):
        pltpu.async_copy(table_hbm.at[idx_v.at[j]], rows_v.at[j], gsems[j])

    def step(h, j, static_tail=False):
        pltpu.make_async_copy(
            table_hbm.at[idx_v.at[h]], rows_v.at[j], gsems[j]).wait()

        def _wait_out():
            pltpu.make_async_copy(
                tr_v.at[j, :, :, pl.ds(0, BT)],
                out_hbm.at[0, :, wid], osems[j]).wait()

        if static_tail:
            _wait_out()
        else:
            pl.when(h >= 6)(_wait_out)

        rows_j = rows_v.at[j]

        @plsc.parallel_loop(0, BT, unroll=4)
        def _transpose(r):
            rv = jnp.full((16,), r, jnp.int32)
            for c in range(4):
                v = rows_j[r, pl.ds(16 * c, 16)]
                plsc.store_scatter(
                    tr_v.at[j], [egvecs[c], esvecs[c], rv], v)

        pltpu.async_copy(
            tr_v.at[j, :, :, pl.ds(0, BT)],
            out_hbm.at[h, :, wid], osems[j])

        if not static_tail:
            @pl.when(h < HIST_LEN - 6)
            def _prefetch():
                pltpu.async_copy(
                    table_hbm.at[idx_v.at[h + 6]], rows_v.at[j], gsems[j])

    def block(b, carry):
        for j in range(6):
            step(b * 6 + j, j)
        return carry

    lax.fori_loop(0, (HIST_LEN // 6), block, None)
    for j in range(2):
        step(HIST_LEN - 2 + j, j, static_tail=True)
    for j in range(6):
        pltpu.make_async_copy(
            tr_v.at[j, :, :, pl.ds(0, BT)],
            out_hbm.at[0, :, wid], osems[j]).wait()


def kernel(X, table):
    table_p = jnp.pad(table, ((0, 0), (0, EMBED_DIM))).reshape(
        2 * VOCAB, EMBED_DIM)
    q = _sc_gather(table_p, X.astype(jnp.int32))
    return q.transpose(2, 4, 0, 1, 3).reshape(BATCH, HIST_LEN, EMBED_DIM)
